# Initial kernel scaffold; baseline (speedup 1.0000x reference)
import jax
import jax.numpy as jnp
from jax.experimental import pallas as pl


def kernel(x, edge_index, batch, W_in, b_in, W_msg, b_msg, W_mr, W_mz, W_mh, W_hr, b_hr, W_hz, b_hz, W_hh, b_hh, W_out, b_out):
    raise NotImplementedError("write your pallas kernel here")


# speedup vs baseline: 4.0523x; 4.0523x over previous
import functools

import jax
import jax.numpy as jnp
from jax.experimental import pallas as pl
from jax.experimental.pallas import tpu as pltpu

D = 256
G = 64
ROUNDS = 4
BLK = 1024


def _gru_kernel(x_ref, b_ref, w_in_ref, b_in_ref, w_scat_ref, b_scat_ref,
                w_mcat_ref, w_hh_ref, b_hh_ref, w_out_ref, b_out_ref,
                out_ref, acc_ref, *, grid):
    i = pl.program_id(0)
    f32 = jnp.float32

    state = jax.nn.relu(
        jnp.dot(x_ref[...], w_in_ref[...], preferred_element_type=f32)
        + b_in_ref[...])

    for r in range(ROUNDS):
        scat = (jnp.dot(state, w_scat_ref[r], preferred_element_type=f32)
                + b_scat_ref[r])
        message = jax.nn.relu(scat[:, :D])
        mcat = jnp.dot(message, w_mcat_ref[...], preferred_element_type=f32)
        rg = jax.nn.sigmoid(mcat[:, :D] + scat[:, D:2 * D])
        zg = jax.nn.sigmoid(mcat[:, D:2 * D] + scat[:, 2 * D:])
        h = jnp.tanh(mcat[:, 2 * D:]
                     + jnp.dot(rg * state, w_hh_ref[...],
                               preferred_element_type=f32)
                     + b_hh_ref[...])
        state = zg * h + (1.0 - zg) * state

    ids = b_ref[0]
    onehot = (jax.lax.broadcasted_iota(jnp.int32, (G, BLK), 0)
              == ids).astype(f32)
    gs = jnp.dot(onehot, state, preferred_element_type=f32)

    @pl.when(i == 0)
    def _init():
        acc_ref[...] = gs

    @pl.when(i > 0)
    def _accum():
        acc_ref[...] += gs

    @pl.when(i == grid - 1)
    def _finish():
        out_ref[...] = (jnp.dot(acc_ref[...], w_out_ref[...],
                                preferred_element_type=f32)
                        + b_out_ref[...])


def kernel(x, edge_index, batch, W_in, b_in, W_msg, b_msg, W_mr, W_mz, W_mh,
           W_hr, b_hr, W_hz, b_hz, W_hh, b_hh, W_out, b_out):
    del edge_index
    n = x.shape[0]
    m = W_out.shape[1]
    grid = pl.cdiv(n, BLK)
    n_pad = grid * BLK - n

    if n_pad:
        x = jnp.pad(x, ((0, n_pad), (0, 0)))
    batch32 = batch.astype(jnp.int32)
    if n_pad:
        batch32 = jnp.pad(batch32, (0, n_pad), constant_values=G)
    batch32 = batch32.reshape(grid, 1, BLK)

    w_scat = jnp.concatenate(
        [W_msg, jnp.broadcast_to(W_hr[None], (ROUNDS, D, D)),
         jnp.broadcast_to(W_hz[None], (ROUNDS, D, D))], axis=2)
    b_scat = jnp.concatenate(
        [b_msg, jnp.broadcast_to(b_hr[None], (ROUNDS, D)),
         jnp.broadcast_to(b_hz[None], (ROUNDS, D))], axis=1)
    b_scat = b_scat.reshape(ROUNDS, 1, 3 * D)
    w_mcat = jnp.concatenate([W_mr, W_mz, W_mh], axis=1)

    const = lambda *zeros: (lambda i: zeros)
    out = pl.pallas_call(
        functools.partial(_gru_kernel, grid=grid),
        grid=(grid,),
        in_specs=[
            pl.BlockSpec((BLK, D), lambda i: (i, 0)),
            pl.BlockSpec((1, 1, BLK), lambda i: (i, 0, 0)),
            pl.BlockSpec((D, D), const(0, 0)),
            pl.BlockSpec((1, D), const(0, 0)),
            pl.BlockSpec((ROUNDS, D, 3 * D), const(0, 0, 0)),
            pl.BlockSpec((ROUNDS, 1, 3 * D), const(0, 0, 0)),
            pl.BlockSpec((D, 3 * D), const(0, 0)),
            pl.BlockSpec((D, D), const(0, 0)),
            pl.BlockSpec((1, D), const(0, 0)),
            pl.BlockSpec((D, m), const(0, 0)),
            pl.BlockSpec((1, m), const(0, 0)),
        ],
        out_specs=pl.BlockSpec((G, m), const(0, 0)),
        out_shape=jax.ShapeDtypeStruct((G, m), jnp.float32),
        scratch_shapes=[pltpu.VMEM((G, D), jnp.float32)],
        compiler_params=pltpu.CompilerParams(
            dimension_semantics=("arbitrary",)),
    )(x, batch32, W_in, b_in.reshape(1, D), w_scat, b_scat, w_mcat,
      W_hh, b_hh.reshape(1, D), W_out, b_out.reshape(1, m))
    return out


# speedup vs baseline: 4.7249x; 1.1660x over previous
import functools

import jax
import jax.numpy as jnp
from jax.experimental import pallas as pl
from jax.experimental.pallas import tpu as pltpu

D = 256
G = 64
ROUNDS = 4
BLK = 2000


def _gru_kernel(x_ref, b_ref, w_in_ref, b_in_ref, w_scat_ref, b_scat_ref,
                w_mcat_ref, w_hh_ref, b_hh_ref, w_out_ref, b_out_ref,
                out_ref, acc_ref, *, grid):
    i = pl.program_id(0)
    f32 = jnp.float32

    state = jax.nn.relu(
        jnp.dot(x_ref[...], w_in_ref[...], preferred_element_type=f32)
        + b_in_ref[...])

    for r in range(ROUNDS):
        scat = (jnp.dot(state, w_scat_ref[r], preferred_element_type=f32)
                + b_scat_ref[r])
        message = jax.nn.relu(scat[:, :D])
        mcat = jnp.dot(message, w_mcat_ref[...], preferred_element_type=f32)
        rg = jax.nn.sigmoid(mcat[:, :D] + scat[:, D:2 * D])
        zg = jax.nn.sigmoid(mcat[:, D:2 * D] + scat[:, 2 * D:])
        h = jnp.tanh(mcat[:, 2 * D:]
                     + jnp.dot(rg * state, w_hh_ref[...],
                               preferred_element_type=f32)
                     + b_hh_ref[...])
        state = zg * h + (1.0 - zg) * state

    ids = b_ref[0]
    onehot = (jax.lax.broadcasted_iota(jnp.int32, (G, BLK), 0)
              == ids).astype(f32)
    gs = jnp.dot(onehot, state, preferred_element_type=f32)

    @pl.when(i == 0)
    def _init():
        acc_ref[...] = gs

    @pl.when(i > 0)
    def _accum():
        acc_ref[...] += gs

    @pl.when(i == grid - 1)
    def _finish():
        out_ref[...] = (jnp.dot(acc_ref[...], w_out_ref[...],
                                preferred_element_type=f32)
                        + b_out_ref[...])


def kernel(x, edge_index, batch, W_in, b_in, W_msg, b_msg, W_mr, W_mz, W_mh,
           W_hr, b_hr, W_hz, b_hz, W_hh, b_hh, W_out, b_out):
    del edge_index
    n = x.shape[0]
    m = W_out.shape[1]
    grid = pl.cdiv(n, BLK)
    n_pad = grid * BLK - n

    if n_pad:
        x = jnp.pad(x, ((0, n_pad), (0, 0)))
    batch32 = batch.astype(jnp.int32)
    if n_pad:
        batch32 = jnp.pad(batch32, (0, n_pad), constant_values=G)
    batch32 = batch32.reshape(grid, 1, BLK)

    w_scat = jnp.concatenate(
        [W_msg, jnp.broadcast_to(W_hr[None], (ROUNDS, D, D)),
         jnp.broadcast_to(W_hz[None], (ROUNDS, D, D))], axis=2)
    b_scat = jnp.concatenate(
        [b_msg, jnp.broadcast_to(b_hr[None], (ROUNDS, D)),
         jnp.broadcast_to(b_hz[None], (ROUNDS, D))], axis=1)
    b_scat = b_scat.reshape(ROUNDS, 1, 3 * D)
    w_mcat = jnp.concatenate([W_mr, W_mz, W_mh], axis=1)

    const = lambda *zeros: (lambda i: zeros)
    out = pl.pallas_call(
        functools.partial(_gru_kernel, grid=grid),
        grid=(grid,),
        in_specs=[
            pl.BlockSpec((BLK, D), lambda i: (i, 0)),
            pl.BlockSpec((1, 1, BLK), lambda i: (i, 0, 0)),
            pl.BlockSpec((D, D), const(0, 0)),
            pl.BlockSpec((1, D), const(0, 0)),
            pl.BlockSpec((ROUNDS, D, 3 * D), const(0, 0, 0)),
            pl.BlockSpec((ROUNDS, 1, 3 * D), const(0, 0, 0)),
            pl.BlockSpec((D, 3 * D), const(0, 0)),
            pl.BlockSpec((D, D), const(0, 0)),
            pl.BlockSpec((1, D), const(0, 0)),
            pl.BlockSpec((D, m), const(0, 0)),
            pl.BlockSpec((1, m), const(0, 0)),
        ],
        out_specs=pl.BlockSpec((G, m), const(0, 0)),
        out_shape=jax.ShapeDtypeStruct((G, m), jnp.float32),
        scratch_shapes=[pltpu.VMEM((G, D), jnp.float32)],
        compiler_params=pltpu.CompilerParams(
            dimension_semantics=("arbitrary",)),
    )(x, batch32, W_in, b_in.reshape(1, D), w_scat, b_scat, w_mcat,
      W_hh, b_hh.reshape(1, D), W_out, b_out.reshape(1, m))
    return out
